# baseline (speedup 1.0000x reference)
import functools

import jax
import jax.numpy as jnp
from jax import lax
from jax.experimental import pallas as pl
from jax.experimental.pallas import tpu as pltpu
from jax.experimental.pallas import tpu_sc as plsc

K = 512
D = 4096
NC = 2
NS = 16
NW = NC * NS
ROWS_PER_W = K // NW

_mesh = plsc.VectorSubcoreMesh(core_axis_name="c", subcore_axis_name="s")


@functools.partial(
    pl.kernel,
    mesh=_mesh,
    out_type=jax.ShapeDtypeStruct((K, D), jnp.float32),
    scratch_types=[pltpu.VMEM((ROWS_PER_W, D), jnp.float32)],
)
def _sc_copy(table_hbm, out_hbm, buf):
    wid = lax.axis_index("s") * NC + lax.axis_index("c")
    base = wid * ROWS_PER_W
    pltpu.sync_copy(table_hbm.at[pl.ds(base, ROWS_PER_W)], buf)
    pltpu.sync_copy(buf, out_hbm.at[pl.ds(base, ROWS_PER_W)])


def kernel(embedding_weight):
    return _sc_copy(embedding_weight)
